# speedup vs baseline: 7.4919x; 7.4919x over previous
import jax
import jax.numpy as jnp
from jax.experimental import pallas as pl
from jax.experimental.pallas import tpu as pltpu

N = 100000
F = 128
G = 64
EPS = 1e-06
R = 2000
NB = N // R


def _body(x_ref, b_ref, w_ref, bias_ref, out_ref, acc_ref, cnt_ref, inv_ref):
    p = pl.program_id(0)
    i = pl.program_id(1)

    @pl.when(jnp.logical_and(p == 0, i == 0))
    def _init():
        acc_ref[...] = jnp.zeros_like(acc_ref)
        cnt_ref[...] = jnp.zeros_like(cnt_ref)

    b = b_ref[0, 0, :]

    @pl.when(p == 0)
    def _accum():
        x = x_ref[...]
        xsq = x * x
        iota_g = jax.lax.broadcasted_iota(jnp.int32, (G, R), 0)
        onehot = (iota_g == b[None, :]).astype(jnp.float32)
        acc_ref[...] += jnp.dot(onehot, xsq,
                                preferred_element_type=jnp.float32)
        cnt_ref[...] += jnp.broadcast_to(
            jnp.sum(onehot, axis=1, keepdims=True), (G, F))

    @pl.when(jnp.logical_and(p == 1, i == 0))
    def _mk_inv():
        mean_sq = acc_ref[...] / jnp.maximum(cnt_ref[...], 1.0)
        inv_ref[...] = jax.lax.rsqrt(mean_sq + EPS)

    @pl.when(p == 1)
    def _normalize():
        x = x_ref[...]
        iota_g2 = jax.lax.broadcasted_iota(jnp.int32, (R, G), 1)
        onehot2 = (iota_g2 == b[:, None]).astype(jnp.float32)
        inv = jnp.dot(onehot2, inv_ref[...],
                      preferred_element_type=jnp.float32)
        out_ref[...] = x * inv * w_ref[0, :] + bias_ref[0, :]


def kernel(x, batch, weight, bias):
    b3 = batch.astype(jnp.int32).reshape(NB, 1, R)
    w2 = weight.reshape(1, F)
    bias2 = bias.reshape(1, F)
    return pl.pallas_call(
        _body,
        grid=(2, NB),
        in_specs=[
            pl.BlockSpec((R, F), lambda p, i: (i, 0)),
            pl.BlockSpec((1, 1, R), lambda p, i: (i, 0, 0)),
            pl.BlockSpec((1, F), lambda p, i: (0, 0)),
            pl.BlockSpec((1, F), lambda p, i: (0, 0)),
        ],
        out_specs=pl.BlockSpec((R, F), lambda p, i: (jnp.where(p == 0, 0, i), 0)),
        out_shape=jax.ShapeDtypeStruct((N, F), jnp.float32),
        scratch_shapes=[
            pltpu.VMEM((G, F), jnp.float32),
            pltpu.VMEM((G, F), jnp.float32),
            pltpu.VMEM((G, F), jnp.float32),
        ],
    )(x, b3, w2, bias2)
